# Initial kernel scaffold; baseline (speedup 1.0000x reference)
import jax
import jax.numpy as jnp
from jax.experimental import pallas as pl


def kernel(adjacency_row, adjacency_col, adjacency_values, embedding, weights):
    raise NotImplementedError("write your pallas kernel here")


# speedup vs baseline: 3.7575x; 3.7575x over previous
import functools

import jax
import jax.numpy as jnp
from jax import lax
from jax.experimental import pallas as pl
from jax.experimental.pallas import tpu as pltpu
from jax.experimental.pallas import tpu_sc as plsc

N = 10000
E = 320000
D = 128
NC = 2
NS = 16
NW = NC * NS
EPW = E // NW
CH = 80
NCHUNK = EPW // CH
ACC_N = 10240
RPT = ACC_N // NS
ZCOPIES = RPT // CH

_MESH = plsc.VectorSubcoreMesh(core_axis_name="c", subcore_axis_name="s")


def _spmm_body(y_hbm, row_hbm, col_hbm, val_hbm, out_hbm,
               acc, rowc, colc, valc, rows, sem):
    c = lax.axis_index("c")
    s = lax.axis_index("s")
    wid = c * NS + s

    zero = jnp.zeros((16,), jnp.float32)

    def zb(i, carry):
        for j in range(8):
            rows[i, pl.ds(j * 16, 16)] = zero
        return carry

    lax.fori_loop(0, CH, zb, 0)
    for t in range(ZCOPIES):
        pltpu.sync_copy(rows, acc.at[pl.ds(s * RPT + t * CH, CH)])
    plsc.subcore_barrier()

    ebase = wid * EPW

    def chunk(k, carry):
        base = pl.multiple_of(ebase + k * CH, 8)
        pltpu.sync_copy(row_hbm.at[pl.ds(base, CH)], rowc)
        pltpu.sync_copy(col_hbm.at[pl.ds(base, CH)], colc)
        pltpu.sync_copy(val_hbm.at[pl.ds(base, CH)], valc)
        pltpu.async_copy(y_hbm.at[colc], rows, sem).wait()

        def edge_group(g, inner):
            vvec = valc[pl.ds(g * 16, 16)]
            for l in range(16):
                v = vvec[l]
                e = g * 16 + l
                for j in range(8):
                    rows[e, pl.ds(j * 16, 16)] = rows[e, pl.ds(j * 16, 16)] * v
            return inner

        lax.fori_loop(0, CH // 16, edge_group, 0)
        pltpu.sync_copy(rows, acc.at[rowc], add=True)
        return carry

    lax.fori_loop(0, NCHUNK, chunk, 0)
    plsc.subcore_barrier()

    pltpu.sync_copy(acc.at[pl.ds(s * RPT, RPT)],
                    out_hbm.at[c, pl.ds(s * RPT, RPT)])


_spmm = pl.kernel(
    _spmm_body,
    out_type=jax.ShapeDtypeStruct((NC, ACC_N, D), jnp.float32),
    mesh=_MESH,
    scratch_types=[
        pltpu.VMEM_SHARED((ACC_N, D), jnp.float32),
        pltpu.VMEM((CH,), jnp.int32),
        pltpu.VMEM((CH,), jnp.int32),
        pltpu.VMEM((CH,), jnp.float32),
        pltpu.VMEM((CH, D), jnp.float32),
        pltpu.SemaphoreType.DMA,
    ],
)


ROWS_BLK = 1000
GRID = N // ROWS_BLK


def _gemm0_body(x_ref, w_ref, y_ref):
    y_ref[...] = jnp.dot(x_ref[...], w_ref[...].T,
                         preferred_element_type=jnp.float32)


_gemm0 = pl.pallas_call(
    _gemm0_body,
    grid=(GRID,),
    in_specs=[
        pl.BlockSpec((ROWS_BLK, D), lambda i: (i, 0)),
        pl.BlockSpec((D, D), lambda i: (0, 0)),
    ],
    out_specs=pl.BlockSpec((ROWS_BLK, D), lambda i: (i, 0)),
    out_shape=jax.ShapeDtypeStruct((N, D), jnp.float32),
)


def _gemm_mid_body(p_ref, w_ref, x_ref, y_ref):
    x = p_ref[0] + p_ref[1]
    x_ref[...] = x
    y_ref[...] = jnp.dot(x, w_ref[...].T, preferred_element_type=jnp.float32)


_gemm_mid = pl.pallas_call(
    _gemm_mid_body,
    grid=(GRID,),
    in_specs=[
        pl.BlockSpec((NC, ROWS_BLK, D), lambda i: (0, i, 0)),
        pl.BlockSpec((D, D), lambda i: (0, 0)),
    ],
    out_specs=[
        pl.BlockSpec((ROWS_BLK, D), lambda i: (i, 0)),
        pl.BlockSpec((ROWS_BLK, D), lambda i: (i, 0)),
    ],
    out_shape=[
        jax.ShapeDtypeStruct((N, D), jnp.float32),
        jax.ShapeDtypeStruct((N, D), jnp.float32),
    ],
)


def _normed(x):
    nrm = jnp.sqrt(jnp.sum(x * x, axis=-1, keepdims=True))
    return x / jnp.maximum(nrm, 1e-12)


def _final_body(x0_ref, x1_ref, x2_ref, p_ref, o_ref):
    x3 = p_ref[0] + p_ref[1]
    o_ref[...] = 0.25 * (x0_ref[...] + _normed(x1_ref[...])
                         + _normed(x2_ref[...]) + _normed(x3))


_final = pl.pallas_call(
    _final_body,
    grid=(GRID,),
    in_specs=[
        pl.BlockSpec((ROWS_BLK, D), lambda i: (i, 0)),
        pl.BlockSpec((ROWS_BLK, D), lambda i: (i, 0)),
        pl.BlockSpec((ROWS_BLK, D), lambda i: (i, 0)),
        pl.BlockSpec((NC, ROWS_BLK, D), lambda i: (0, i, 0)),
    ],
    out_specs=pl.BlockSpec((ROWS_BLK, D), lambda i: (i, 0)),
    out_shape=jax.ShapeDtypeStruct((N, D), jnp.float32),
)


def kernel(adjacency_row, adjacency_col, adjacency_values, embedding, weights):
    y0 = _gemm0(embedding, weights[0])
    p1 = _spmm(y0, adjacency_row, adjacency_col, adjacency_values)
    x1, y1 = _gemm_mid(p1, weights[1])
    p2 = _spmm(y1, adjacency_row, adjacency_col, adjacency_values)
    x2, y2 = _gemm_mid(p2, weights[2])
    p3 = _spmm(y2, adjacency_row, adjacency_col, adjacency_values)
    return _final(embedding, x1, x2, p3)
